# Initial kernel scaffold; baseline (speedup 1.0000x reference)
import jax
import jax.numpy as jnp
from jax.experimental import pallas as pl


def kernel(x, edge_index, W1a, b1a, W1b, b1b, W4a, b4a, W4b, b4b):
    raise NotImplementedError("write your pallas kernel here")


# speedup vs baseline: 4.6292x; 4.6292x over previous
import functools

import jax
import jax.numpy as jnp
from jax import lax
from jax.experimental import pallas as pl
from jax.experimental.pallas import tpu as pltpu
from jax.experimental.pallas import tpu_sc as plsc

N = 10000
E = 320000
D = 128
NORM_SCALE = 20.0

_NC = 2
_NS = 16
_NW = _NC * _NS
_EPT = E // _NW
_CHUNK = 80
_NCHUNK = _EPT // _CHUNK
_CP_TILES = 10
_ROWS_PT = N // _CP_TILES


def _sc_scatter(feat, zeros, src, dst):
  mesh = plsc.VectorSubcoreMesh(core_axis_name="c", subcore_axis_name="s")

  @functools.partial(
      pl.kernel,
      out_type=jax.ShapeDtypeStruct((2 * N, D), jnp.float32),
      mesh=mesh,
      scratch_types=[
          pltpu.VMEM((_CHUNK,), jnp.int32),
          pltpu.VMEM((_CHUNK,), jnp.int32),
          pltpu.VMEM((_CHUNK, D), jnp.float32),
          pltpu.VMEM_SHARED((N, D), jnp.float32),
          pltpu.SemaphoreType.DMA,
      ],
  )
  def k(feat_hbm, zero_hbm, src_hbm, dst_hbm, out_hbm, sidx, didx, rows, acc,
        sem):
    c = lax.axis_index("c")
    s = lax.axis_index("s")
    wid = s * _NC + c
    r0 = s * _ROWS_PT

    @pl.when(jnp.logical_and(s < _CP_TILES, c == 0))
    def _():
      pltpu.sync_copy(feat_hbm.at[pl.ds(r0, _ROWS_PT)],
                      acc.at[pl.ds(r0, _ROWS_PT)])

    @pl.when(jnp.logical_and(s < _CP_TILES, c != 0))
    def _():
      pltpu.sync_copy(zero_hbm.at[pl.ds(r0, _ROWS_PT)],
                      acc.at[pl.ds(r0, _ROWS_PT)])

    plsc.subcore_barrier()

    base0 = wid * _EPT

    def body(i, carry):
      b = base0 + i * _CHUNK
      pltpu.sync_copy(src_hbm.at[pl.ds(b, _CHUNK)], sidx)
      pltpu.sync_copy(dst_hbm.at[pl.ds(b, _CHUNK)], didx)
      pltpu.async_copy(feat_hbm.at[sidx], rows, sem).wait()
      pltpu.sync_copy(rows, acc.at[didx], add=True)
      return carry

    lax.fori_loop(0, _NCHUNK, body, 0)
    plsc.subcore_barrier()

    @pl.when(s < _CP_TILES)
    def _():
      pltpu.sync_copy(acc.at[pl.ds(r0, _ROWS_PT)],
                      out_hbm.at[pl.ds(c * N + r0, _ROWS_PT)])

  return k(feat, zeros, src, dst)


_BLK = 1000


def _mlp_norm_body(p_ref, wa_ref, ba_ref, wb_ref, bb_ref, scaled_ref,
                   colsum_ref):
  i = pl.program_id(0)
  h = p_ref[0] + p_ref[1]
  t = jnp.maximum(
      jnp.dot(h, wa_ref[...], preferred_element_type=jnp.float32)
      + ba_ref[...], 0.0)
  l1 = jnp.dot(t, wb_ref[...], preferred_element_type=jnp.float32) + bb_ref[...]
  rn = jnp.sqrt(1e-6 + jnp.sum(l1 * l1, axis=1, keepdims=True))
  scaled_ref[...] = NORM_SCALE * l1 / rn
  csum = jnp.sum(l1, axis=0, keepdims=True)

  @pl.when(i == 0)
  def _():
    colsum_ref[...] = csum

  @pl.when(i > 0)
  def _():
    colsum_ref[...] += csum


def _tc_mlp_norm(parts, wa, ba, wb, bb):
  grid = (N // _BLK,)
  return pl.pallas_call(
      _mlp_norm_body,
      grid=grid,
      in_specs=[
          pl.BlockSpec((2, _BLK, D), lambda i: (0, i, 0)),
          pl.BlockSpec((D, D), lambda i: (0, 0)),
          pl.BlockSpec((1, D), lambda i: (0, 0)),
          pl.BlockSpec((D, D), lambda i: (0, 0)),
          pl.BlockSpec((1, D), lambda i: (0, 0)),
      ],
      out_specs=[
          pl.BlockSpec((_BLK, D), lambda i: (i, 0)),
          pl.BlockSpec((1, D), lambda i: (0, 0)),
      ],
      out_shape=[
          jax.ShapeDtypeStruct((N, D), jnp.float32),
          jax.ShapeDtypeStruct((1, D), jnp.float32),
      ],
  )(parts, wa, ba, wb, bb)


def _finish_body(scaled_ref, colsum_ref, out_ref):
  out_ref[...] = jnp.maximum(
      scaled_ref[...] - colsum_ref[...] * (1.0 / N), 0.0)


def _tc_finish_norm(scaled, colsum):
  return pl.pallas_call(
      _finish_body,
      out_shape=jax.ShapeDtypeStruct((N, D), jnp.float32),
  )(scaled, colsum)


def _mlp_body(p_ref, wa_ref, ba_ref, wb_ref, bb_ref, out_ref):
  h = p_ref[0] + p_ref[1]
  t = jnp.maximum(
      jnp.dot(h, wa_ref[...], preferred_element_type=jnp.float32)
      + ba_ref[...], 0.0)
  out_ref[...] = (
      jnp.dot(t, wb_ref[...], preferred_element_type=jnp.float32)
      + bb_ref[...])


def _tc_mlp(parts, wa, ba, wb, bb):
  grid = (N // _BLK,)
  return pl.pallas_call(
      _mlp_body,
      grid=grid,
      in_specs=[
          pl.BlockSpec((2, _BLK, D), lambda i: (0, i, 0)),
          pl.BlockSpec((D, D), lambda i: (0, 0)),
          pl.BlockSpec((1, D), lambda i: (0, 0)),
          pl.BlockSpec((D, D), lambda i: (0, 0)),
          pl.BlockSpec((1, D), lambda i: (0, 0)),
      ],
      out_specs=pl.BlockSpec((_BLK, D), lambda i: (i, 0)),
      out_shape=jax.ShapeDtypeStruct((N, D), jnp.float32),
  )(parts, wa, ba, wb, bb)


def kernel(x, edge_index, W1a, b1a, W1b, b1b, W4a, b4a, W4b, b4b):
  src = edge_index[0]
  dst = edge_index[1]
  zeros = jnp.zeros((N, D), jnp.float32)
  b1a2 = b1a.reshape(1, D)
  b1b2 = b1b.reshape(1, D)
  b4a2 = b4a.reshape(1, D)
  b4b2 = b4b.reshape(1, D)

  p = _sc_scatter(x, zeros, src, dst).reshape(2, N, D)
  scaled, colsum = _tc_mlp_norm(p, W1a, b1a2, W1b, b1b2)
  l1n = _tc_finish_norm(scaled, colsum)
  q = _sc_scatter(l1n, zeros, src, dst).reshape(2, N, D)
  return _tc_mlp(q, W4a, b4a2, W4b, b4b2)


# speedup vs baseline: 10.5442x; 2.2777x over previous
import functools

import jax
import jax.numpy as jnp
from jax import lax
from jax.experimental import pallas as pl
from jax.experimental.pallas import tpu as pltpu
from jax.experimental.pallas import tpu_sc as plsc

N = 10000
E = 320000
D = 128
NORM_SCALE = 20.0

_NC = 2
_NS = 16
_NW = _NC * _NS
_EPT = E // _NW
_CHUNK = 40
_NCHUNK = _EPT // _CHUNK
_M = 6
_DS = 2
_DG = 2
_DI = 4
assert _DI + _DS == _M and _DG < _DI
_CP_TILES = 10
_ROWS_PT = N // _CP_TILES


def _sc_scatter(feat, zeros, idx4):
  mesh = plsc.VectorSubcoreMesh(core_axis_name="c", subcore_axis_name="s")

  @functools.partial(
      pl.kernel,
      out_type=jax.ShapeDtypeStruct((2 * N, D), jnp.float32),
      mesh=mesh,
      scratch_types=[
          pltpu.VMEM((_M, 2, _CHUNK), jnp.int32),
          pltpu.VMEM((_M, _CHUNK, D), jnp.float32),
          pltpu.VMEM_SHARED((N, D), jnp.float32),
          pltpu.SemaphoreType.DMA,
          pltpu.SemaphoreType.DMA,
          pltpu.SemaphoreType.DMA,
      ],
  )
  def k(feat_hbm, zero_hbm, idx_hbm, out_hbm, idxb, rows, acc,
        sem_i, sem_g, sem_s):
    c = lax.axis_index("c")
    s = lax.axis_index("s")
    wid = s * _NC + c
    r0 = s * _ROWS_PT

    def fire_idx(i, m):
      pltpu.async_copy(idx_hbm.at[wid].at[i], idxb.at[m], sem_i)

    def drain_idx(m):
      pltpu.make_async_copy(idx_hbm.at[0].at[0], idxb.at[m], sem_i).wait()

    def fire_gather(i, m):
      pltpu.async_copy(feat_hbm.at[idxb.at[m].at[0]], rows.at[m], sem_g)

    def drain_gather(m):
      pltpu.make_async_copy(feat_hbm.at[pl.ds(0, _CHUNK)], rows.at[m],
                            sem_g).wait()

    def fire_scatter(i, m):
      pltpu.async_copy(rows.at[m], acc.at[idxb.at[m].at[1]], sem_s, add=True)

    def drain_scatter(m):
      pltpu.make_async_copy(feat_hbm.at[pl.ds(0, _CHUNK)], rows.at[m],
                            sem_s).wait()

    for j in range(_DI):
      fire_idx(j, j)
    for j in range(_DG):
      drain_idx(j)
      fire_gather(j, j)

    @pl.when(jnp.logical_and(s < _CP_TILES, c == 0))
    def _():
      pltpu.sync_copy(feat_hbm.at[pl.ds(r0, _ROWS_PT)],
                      acc.at[pl.ds(r0, _ROWS_PT)])

    @pl.when(jnp.logical_and(s < _CP_TILES, c != 0))
    def _():
      pltpu.sync_copy(zero_hbm.at[pl.ds(r0, _ROWS_PT)],
                      acc.at[pl.ds(r0, _ROWS_PT)])

    plsc.subcore_barrier()

    def step(i, guard_lo=True, guard_hi=True):
      m = i % _M
      if guard_lo:
        drain_scatter((i - _DS) % _M)
      if guard_hi:
        fire_idx(i + _DI, (i + _DI) % _M)
        drain_idx((i + _DG) % _M)
        fire_gather(i + _DG, (i + _DG) % _M)
      drain_gather(m)
      fire_scatter(i, m)

    for i in range(_DS):
      step(i, guard_lo=False)

    _STEADY_LO = _DS
    _STEADY_HI = _NCHUNK - _DI - ((_NCHUNK - _DI - _DS) % _M)

    def steady(t, carry):
      i0 = _STEADY_LO + t * _M
      for kk in range(_M):
        i = i0 + kk
        m = (_STEADY_LO + kk) % _M
        drain_scatter((m - _DS) % _M)
        fire_idx(i + _DI, (m + _DI) % _M)
        drain_idx((m + _DG) % _M)
        fire_gather(i + _DG, (m + _DG) % _M)
        drain_gather(m)
        fire_scatter(i, m)
      return carry

    lax.fori_loop(0, (_STEADY_HI - _STEADY_LO) // _M, steady, 0)

    for i in range(_STEADY_HI, _NCHUNK):
      m = i % _M
      drain_scatter((i - _DS) % _M)
      if i + _DI < _NCHUNK:
        fire_idx(i + _DI, (i + _DI) % _M)
      if i + _DG < _NCHUNK:
        drain_idx((i + _DG) % _M)
        fire_gather(i + _DG, (i + _DG) % _M)
      drain_gather(m)
      fire_scatter(i, m)

    for i in range(_NCHUNK - _DS, _NCHUNK):
      drain_scatter(i % _M)

    plsc.subcore_barrier()

    @pl.when(s < _CP_TILES)
    def _():
      pltpu.sync_copy(acc.at[pl.ds(r0, _ROWS_PT)],
                      out_hbm.at[pl.ds(c * N + r0, _ROWS_PT)])

  return k(feat, zeros, idx4)


_BLK = 1000


def _mlp_norm_body(p_ref, wa_ref, ba_ref, wb_ref, bb_ref, scaled_ref,
                   colsum_ref):
  i = pl.program_id(0)
  h = p_ref[0] + p_ref[1]
  t = jnp.maximum(
      jnp.dot(h, wa_ref[...], preferred_element_type=jnp.float32)
      + ba_ref[...], 0.0)
  l1 = jnp.dot(t, wb_ref[...], preferred_element_type=jnp.float32) + bb_ref[...]
  rn = jnp.sqrt(1e-6 + jnp.sum(l1 * l1, axis=1, keepdims=True))
  scaled_ref[...] = NORM_SCALE * l1 / rn
  csum = jnp.sum(l1, axis=0, keepdims=True)

  @pl.when(i == 0)
  def _():
    colsum_ref[...] = csum

  @pl.when(i > 0)
  def _():
    colsum_ref[...] += csum


def _tc_mlp_norm(parts, wa, ba, wb, bb):
  grid = (N // _BLK,)
  return pl.pallas_call(
      _mlp_norm_body,
      grid=grid,
      in_specs=[
          pl.BlockSpec((2, _BLK, D), lambda i: (0, i, 0)),
          pl.BlockSpec((D, D), lambda i: (0, 0)),
          pl.BlockSpec((1, D), lambda i: (0, 0)),
          pl.BlockSpec((D, D), lambda i: (0, 0)),
          pl.BlockSpec((1, D), lambda i: (0, 0)),
      ],
      out_specs=[
          pl.BlockSpec((_BLK, D), lambda i: (i, 0)),
          pl.BlockSpec((1, D), lambda i: (0, 0)),
      ],
      out_shape=[
          jax.ShapeDtypeStruct((N, D), jnp.float32),
          jax.ShapeDtypeStruct((1, D), jnp.float32),
      ],
  )(parts, wa, ba, wb, bb)


def _finish_body(scaled_ref, colsum_ref, out_ref):
  out_ref[...] = jnp.maximum(
      scaled_ref[...] - colsum_ref[...] * (1.0 / N), 0.0)


def _tc_finish_norm(scaled, colsum):
  return pl.pallas_call(
      _finish_body,
      out_shape=jax.ShapeDtypeStruct((N, D), jnp.float32),
  )(scaled, colsum)


def _mlp_body(p_ref, wa_ref, ba_ref, wb_ref, bb_ref, out_ref):
  h = p_ref[0] + p_ref[1]
  t = jnp.maximum(
      jnp.dot(h, wa_ref[...], preferred_element_type=jnp.float32)
      + ba_ref[...], 0.0)
  out_ref[...] = (
      jnp.dot(t, wb_ref[...], preferred_element_type=jnp.float32)
      + bb_ref[...])


def _tc_mlp(parts, wa, ba, wb, bb):
  grid = (N // _BLK,)
  return pl.pallas_call(
      _mlp_body,
      grid=grid,
      in_specs=[
          pl.BlockSpec((2, _BLK, D), lambda i: (0, i, 0)),
          pl.BlockSpec((D, D), lambda i: (0, 0)),
          pl.BlockSpec((1, D), lambda i: (0, 0)),
          pl.BlockSpec((D, D), lambda i: (0, 0)),
          pl.BlockSpec((1, D), lambda i: (0, 0)),
      ],
      out_specs=pl.BlockSpec((_BLK, D), lambda i: (i, 0)),
      out_shape=jax.ShapeDtypeStruct((N, D), jnp.float32),
  )(parts, wa, ba, wb, bb)


def kernel(x, edge_index, W1a, b1a, W1b, b1b, W4a, b4a, W4b, b4b):
  idx4 = jnp.stack(
      [edge_index[0].reshape(_NW, _NCHUNK, _CHUNK),
       edge_index[1].reshape(_NW, _NCHUNK, _CHUNK)], axis=2)
  zeros = jnp.zeros((N, D), jnp.float32)
  b1a2 = b1a.reshape(1, D)
  b1b2 = b1b.reshape(1, D)
  b4a2 = b4a.reshape(1, D)
  b4b2 = b4b.reshape(1, D)

  p = _sc_scatter(x, zeros, idx4).reshape(2, N, D)
  scaled, colsum = _tc_mlp_norm(p, W1a, b1a2, W1b, b1b2)
  l1n = _tc_finish_norm(scaled, colsum)
  q = _sc_scatter(l1n, zeros, idx4).reshape(2, N, D)
  return _tc_mlp(q, W4a, b4a2, W4b, b4b2)


# speedup vs baseline: 11.1182x; 1.0544x over previous
import functools

import jax
import jax.numpy as jnp
from jax import lax
from jax.experimental import pallas as pl
from jax.experimental.pallas import tpu as pltpu
from jax.experimental.pallas import tpu_sc as plsc

N = 10000
E = 320000
D = 128
NORM_SCALE = 20.0

_NC = 2
_NS = 16
_NW = _NC * _NS
_EPT = E // _NW
_CHUNK = 40
_NCHUNK = _EPT // _CHUNK
_M = 9
_DS = 3
_DG = 3
_DI = 6
assert _DI + _DS == _M and _DG < _DI
_CP_TILES = 10
_ROWS_PT = N // _CP_TILES


def _sc_scatter(feat, zeros, idx4):
  mesh = plsc.VectorSubcoreMesh(core_axis_name="c", subcore_axis_name="s")

  @functools.partial(
      pl.kernel,
      out_type=jax.ShapeDtypeStruct((2 * N, D), jnp.float32),
      mesh=mesh,
      scratch_types=[
          pltpu.VMEM((_M, 2, _CHUNK), jnp.int32),
          pltpu.VMEM((_M, _CHUNK, D), jnp.float32),
          pltpu.VMEM_SHARED((N, D), jnp.float32),
          pltpu.SemaphoreType.DMA,
          pltpu.SemaphoreType.DMA,
          pltpu.SemaphoreType.DMA,
      ],
  )
  def k(feat_hbm, zero_hbm, idx_hbm, out_hbm, idxb, rows, acc,
        sem_i, sem_g, sem_s):
    c = lax.axis_index("c")
    s = lax.axis_index("s")
    wid = s * _NC + c
    r0 = s * _ROWS_PT

    def fire_idx(i, m):
      pltpu.async_copy(idx_hbm.at[wid].at[i], idxb.at[m], sem_i)

    def drain_idx(m):
      pltpu.make_async_copy(idx_hbm.at[0].at[0], idxb.at[m], sem_i).wait()

    def fire_gather(i, m):
      pltpu.async_copy(feat_hbm.at[idxb.at[m].at[0]], rows.at[m], sem_g)

    def drain_gather(m):
      pltpu.make_async_copy(feat_hbm.at[pl.ds(0, _CHUNK)], rows.at[m],
                            sem_g).wait()

    def fire_scatter(i, m):
      pltpu.async_copy(rows.at[m], acc.at[idxb.at[m].at[1]], sem_s, add=True)

    def drain_scatter(m):
      pltpu.make_async_copy(feat_hbm.at[pl.ds(0, _CHUNK)], rows.at[m],
                            sem_s).wait()

    for j in range(_DI):
      fire_idx(j, j)
    for j in range(_DG):
      drain_idx(j)
      fire_gather(j, j)

    @pl.when(jnp.logical_and(s < _CP_TILES, c == 0))
    def _():
      pltpu.sync_copy(feat_hbm.at[pl.ds(r0, _ROWS_PT)],
                      acc.at[pl.ds(r0, _ROWS_PT)])

    @pl.when(jnp.logical_and(s < _CP_TILES, c != 0))
    def _():
      pltpu.sync_copy(zero_hbm.at[pl.ds(r0, _ROWS_PT)],
                      acc.at[pl.ds(r0, _ROWS_PT)])

    plsc.subcore_barrier()

    def step(i, guard_lo=True, guard_hi=True):
      m = i % _M
      if guard_lo:
        drain_scatter((i - _DS) % _M)
      if guard_hi:
        fire_idx(i + _DI, (i + _DI) % _M)
        drain_idx((i + _DG) % _M)
        fire_gather(i + _DG, (i + _DG) % _M)
      drain_gather(m)
      fire_scatter(i, m)

    for i in range(_DS):
      step(i, guard_lo=False)

    _STEADY_LO = _DS
    _STEADY_HI = _NCHUNK - _DI - ((_NCHUNK - _DI - _DS) % _M)

    def steady(t, carry):
      i0 = _STEADY_LO + t * _M
      for kk in range(_M):
        i = i0 + kk
        m = (_STEADY_LO + kk) % _M
        drain_scatter((m - _DS) % _M)
        fire_idx(i + _DI, (m + _DI) % _M)
        drain_idx((m + _DG) % _M)
        fire_gather(i + _DG, (m + _DG) % _M)
        drain_gather(m)
        fire_scatter(i, m)
      return carry

    lax.fori_loop(0, (_STEADY_HI - _STEADY_LO) // _M, steady, 0)

    for i in range(_STEADY_HI, _NCHUNK):
      m = i % _M
      drain_scatter((i - _DS) % _M)
      if i + _DI < _NCHUNK:
        fire_idx(i + _DI, (i + _DI) % _M)
      if i + _DG < _NCHUNK:
        drain_idx((i + _DG) % _M)
        fire_gather(i + _DG, (i + _DG) % _M)
      drain_gather(m)
      fire_scatter(i, m)

    for i in range(_NCHUNK - _DS, _NCHUNK):
      drain_scatter(i % _M)

    plsc.subcore_barrier()

    @pl.when(s < _CP_TILES)
    def _():
      pltpu.sync_copy(acc.at[pl.ds(r0, _ROWS_PT)],
                      out_hbm.at[pl.ds(c * N + r0, _ROWS_PT)])

  return k(feat, zeros, idx4)


_BLK = 1000


def _mlp_norm_body(p_ref, wa_ref, ba_ref, wb_ref, bb_ref, scaled_ref,
                   colsum_ref):
  i = pl.program_id(0)
  h = p_ref[0] + p_ref[1]
  t = jnp.maximum(
      jnp.dot(h, wa_ref[...], preferred_element_type=jnp.float32)
      + ba_ref[...], 0.0)
  l1 = jnp.dot(t, wb_ref[...], preferred_element_type=jnp.float32) + bb_ref[...]
  rn = jnp.sqrt(1e-6 + jnp.sum(l1 * l1, axis=1, keepdims=True))
  scaled_ref[...] = NORM_SCALE * l1 / rn
  csum = jnp.sum(l1, axis=0, keepdims=True)

  @pl.when(i == 0)
  def _():
    colsum_ref[...] = csum

  @pl.when(i > 0)
  def _():
    colsum_ref[...] += csum


def _tc_mlp_norm(parts, wa, ba, wb, bb):
  grid = (N // _BLK,)
  return pl.pallas_call(
      _mlp_norm_body,
      grid=grid,
      in_specs=[
          pl.BlockSpec((2, _BLK, D), lambda i: (0, i, 0)),
          pl.BlockSpec((D, D), lambda i: (0, 0)),
          pl.BlockSpec((1, D), lambda i: (0, 0)),
          pl.BlockSpec((D, D), lambda i: (0, 0)),
          pl.BlockSpec((1, D), lambda i: (0, 0)),
      ],
      out_specs=[
          pl.BlockSpec((_BLK, D), lambda i: (i, 0)),
          pl.BlockSpec((1, D), lambda i: (0, 0)),
      ],
      out_shape=[
          jax.ShapeDtypeStruct((N, D), jnp.float32),
          jax.ShapeDtypeStruct((1, D), jnp.float32),
      ],
  )(parts, wa, ba, wb, bb)


def _finish_body(scaled_ref, colsum_ref, out_ref):
  out_ref[...] = jnp.maximum(
      scaled_ref[...] - colsum_ref[...] * (1.0 / N), 0.0)


def _tc_finish_norm(scaled, colsum):
  return pl.pallas_call(
      _finish_body,
      out_shape=jax.ShapeDtypeStruct((N, D), jnp.float32),
  )(scaled, colsum)


def _mlp_body(p_ref, wa_ref, ba_ref, wb_ref, bb_ref, out_ref):
  h = p_ref[0] + p_ref[1]
  t = jnp.maximum(
      jnp.dot(h, wa_ref[...], preferred_element_type=jnp.float32)
      + ba_ref[...], 0.0)
  out_ref[...] = (
      jnp.dot(t, wb_ref[...], preferred_element_type=jnp.float32)
      + bb_ref[...])


def _tc_mlp(parts, wa, ba, wb, bb):
  grid = (N // _BLK,)
  return pl.pallas_call(
      _mlp_body,
      grid=grid,
      in_specs=[
          pl.BlockSpec((2, _BLK, D), lambda i: (0, i, 0)),
          pl.BlockSpec((D, D), lambda i: (0, 0)),
          pl.BlockSpec((1, D), lambda i: (0, 0)),
          pl.BlockSpec((D, D), lambda i: (0, 0)),
          pl.BlockSpec((1, D), lambda i: (0, 0)),
      ],
      out_specs=pl.BlockSpec((_BLK, D), lambda i: (i, 0)),
      out_shape=jax.ShapeDtypeStruct((N, D), jnp.float32),
  )(parts, wa, ba, wb, bb)


def kernel(x, edge_index, W1a, b1a, W1b, b1b, W4a, b4a, W4b, b4b):
  idx4 = jnp.stack(
      [edge_index[0].reshape(_NW, _NCHUNK, _CHUNK),
       edge_index[1].reshape(_NW, _NCHUNK, _CHUNK)], axis=2)
  zeros = jnp.zeros((N, D), jnp.float32)
  b1a2 = b1a.reshape(1, D)
  b1b2 = b1b.reshape(1, D)
  b4a2 = b4a.reshape(1, D)
  b4b2 = b4b.reshape(1, D)

  p = _sc_scatter(x, zeros, idx4).reshape(2, N, D)
  scaled, colsum = _tc_mlp_norm(p, W1a, b1a2, W1b, b1b2)
  l1n = _tc_finish_norm(scaled, colsum)
  q = _sc_scatter(l1n, zeros, idx4).reshape(2, N, D)
  return _tc_mlp(q, W4a, b4a2, W4b, b4b2)


# speedup vs baseline: 11.2160x; 1.0088x over previous
import functools

import jax
import jax.numpy as jnp
from jax import lax
from jax.experimental import pallas as pl
from jax.experimental.pallas import tpu as pltpu
from jax.experimental.pallas import tpu_sc as plsc

N = 10000
E = 320000
D = 128
NORM_SCALE = 20.0

_NC = 2
_NS = 16
_NW = _NC * _NS
_EPT = E // _NW
_CHUNK = 40
_NCHUNK = _EPT // _CHUNK
_M = 9
_DS = 3
_DG = 3
_DI = 6
assert _DI + _DS == _M and _DG < _DI
_CP_TILES = 10
_ROWS_PT = N // _CP_TILES


def _sc_scatter(feat, zeros, idx4):
  mesh = plsc.VectorSubcoreMesh(core_axis_name="c", subcore_axis_name="s")

  @functools.partial(
      pl.kernel,
      out_type=jax.ShapeDtypeStruct((2 * N, D), jnp.float32),
      mesh=mesh,
      scratch_types=[
          pltpu.VMEM((_M, 2, _CHUNK), jnp.int32),
          pltpu.VMEM((_M, _CHUNK, D), jnp.float32),
          pltpu.VMEM_SHARED((N, D), jnp.float32),
          pltpu.SemaphoreType.DMA,
          pltpu.SemaphoreType.DMA,
          pltpu.SemaphoreType.DMA,
      ],
  )
  def k(feat_hbm, zero_hbm, idx_hbm, out_hbm, idxb, rows, acc,
        sem_i, sem_g, sem_s):
    c = lax.axis_index("c")
    s = lax.axis_index("s")
    wid = s * _NC + c
    r0 = s * _ROWS_PT

    def fire_idx(i, m):
      pltpu.async_copy(idx_hbm.at[wid].at[i], idxb.at[m], sem_i)

    def drain_idx(m):
      pltpu.make_async_copy(idx_hbm.at[0].at[0], idxb.at[m], sem_i).wait()

    def fire_gather(i, m):
      pltpu.async_copy(feat_hbm.at[idxb.at[m].at[0]], rows.at[m], sem_g)

    def drain_gather(m):
      pltpu.make_async_copy(feat_hbm.at[pl.ds(0, _CHUNK)], rows.at[m],
                            sem_g).wait()

    def fire_scatter(i, m):
      pltpu.async_copy(rows.at[m], acc.at[idxb.at[m].at[1]], sem_s, add=True)

    def drain_scatter(m):
      pltpu.make_async_copy(feat_hbm.at[pl.ds(0, _CHUNK)], rows.at[m],
                            sem_s).wait()

    for j in range(_DI):
      fire_idx(j, j)
    for j in range(_DG):
      drain_idx(j)
      fire_gather(j, j)

    @pl.when(jnp.logical_and(s < _CP_TILES, c == 0))
    def _():
      pltpu.sync_copy(feat_hbm.at[pl.ds(r0, _ROWS_PT)],
                      acc.at[pl.ds(r0, _ROWS_PT)])

    @pl.when(jnp.logical_and(s < _CP_TILES, c != 0))
    def _():
      pltpu.sync_copy(zero_hbm.at[pl.ds(r0, _ROWS_PT)],
                      acc.at[pl.ds(r0, _ROWS_PT)])

    plsc.subcore_barrier()

    def step(i, guard_lo=True, guard_hi=True):
      m = i % _M
      if guard_lo:
        drain_scatter((i - _DS) % _M)
      if guard_hi:
        fire_idx(i + _DI, (i + _DI) % _M)
        drain_idx((i + _DG) % _M)
        fire_gather(i + _DG, (i + _DG) % _M)
      drain_gather(m)
      fire_scatter(i, m)

    for i in range(_DS):
      step(i, guard_lo=False)

    _STEADY_LO = _DS
    _STEADY_HI = _NCHUNK - _DI - ((_NCHUNK - _DI - _DS) % _M)

    def steady(t, carry):
      i0 = _STEADY_LO + t * _M
      for kk in range(_M):
        i = i0 + kk
        m = (_STEADY_LO + kk) % _M
        drain_scatter((m - _DS) % _M)
        fire_idx(i + _DI, (m + _DI) % _M)
        drain_idx((m + _DG) % _M)
        fire_gather(i + _DG, (m + _DG) % _M)
        drain_gather(m)
        fire_scatter(i, m)
      return carry

    lax.fori_loop(0, (_STEADY_HI - _STEADY_LO) // _M, steady, 0)

    for i in range(_STEADY_HI, _NCHUNK):
      m = i % _M
      drain_scatter((i - _DS) % _M)
      if i + _DI < _NCHUNK:
        fire_idx(i + _DI, (i + _DI) % _M)
      if i + _DG < _NCHUNK:
        drain_idx((i + _DG) % _M)
        fire_gather(i + _DG, (i + _DG) % _M)
      drain_gather(m)
      fire_scatter(i, m)

    for i in range(_NCHUNK - _DS, _NCHUNK):
      drain_scatter(i % _M)

    plsc.subcore_barrier()

    @pl.when(s < _CP_TILES)
    def _():
      pltpu.sync_copy(acc.at[pl.ds(r0, _ROWS_PT)],
                      out_hbm.at[pl.ds(c * N + r0, _ROWS_PT)])

  return k(feat, zeros, idx4)


_BLK = 1000


_NBLK = N // _BLK


def _mlp_norm_body(p_ref, wa_ref, ba_ref, wb_ref, bb_ref, out_ref,
                   scaled_scr, colsum_scr):
  i = pl.program_id(0)

  @pl.when(i < _NBLK)
  def _():
    h = p_ref[0] + p_ref[1]
    t = jnp.maximum(
        jnp.dot(h, wa_ref[...], preferred_element_type=jnp.float32)
        + ba_ref[...], 0.0)
    l1 = (jnp.dot(t, wb_ref[...], preferred_element_type=jnp.float32)
          + bb_ref[...])
    rn = jnp.sqrt(1e-6 + jnp.sum(l1 * l1, axis=1, keepdims=True))
    scaled_scr[pl.ds(i * _BLK, _BLK), :] = NORM_SCALE * l1 / rn
    csum = jnp.sum(l1, axis=0, keepdims=True)
    colsum_scr[...] = jnp.where(i == 0, csum, colsum_scr[...] + csum)

  @pl.when(i >= _NBLK)
  def _():
    j = i - _NBLK
    out_ref[...] = jnp.maximum(
        scaled_scr[pl.ds(j * _BLK, _BLK), :]
        - colsum_scr[...] * (1.0 / N), 0.0)


def _tc_mlp_norm(parts, wa, ba, wb, bb):
  grid = (2 * _NBLK,)
  return pl.pallas_call(
      _mlp_norm_body,
      grid=grid,
      in_specs=[
          pl.BlockSpec((2, _BLK, D), lambda i: (0, jnp.minimum(i, _NBLK - 1),
                                                0)),
          pl.BlockSpec((D, D), lambda i: (0, 0)),
          pl.BlockSpec((1, D), lambda i: (0, 0)),
          pl.BlockSpec((D, D), lambda i: (0, 0)),
          pl.BlockSpec((1, D), lambda i: (0, 0)),
      ],
      out_specs=pl.BlockSpec((_BLK, D),
                             lambda i: (jnp.maximum(i - _NBLK, 0), 0)),
      out_shape=jax.ShapeDtypeStruct((N, D), jnp.float32),
      scratch_shapes=[
          pltpu.VMEM((N, D), jnp.float32),
          pltpu.VMEM((1, D), jnp.float32),
      ],
  )(parts, wa, ba, wb, bb)


def _mlp_body(p_ref, wa_ref, ba_ref, wb_ref, bb_ref, out_ref):
  h = p_ref[0] + p_ref[1]
  t = jnp.maximum(
      jnp.dot(h, wa_ref[...], preferred_element_type=jnp.float32)
      + ba_ref[...], 0.0)
  out_ref[...] = (
      jnp.dot(t, wb_ref[...], preferred_element_type=jnp.float32)
      + bb_ref[...])


def _tc_mlp(parts, wa, ba, wb, bb):
  grid = (N // _BLK,)
  return pl.pallas_call(
      _mlp_body,
      grid=grid,
      in_specs=[
          pl.BlockSpec((2, _BLK, D), lambda i: (0, i, 0)),
          pl.BlockSpec((D, D), lambda i: (0, 0)),
          pl.BlockSpec((1, D), lambda i: (0, 0)),
          pl.BlockSpec((D, D), lambda i: (0, 0)),
          pl.BlockSpec((1, D), lambda i: (0, 0)),
      ],
      out_specs=pl.BlockSpec((_BLK, D), lambda i: (i, 0)),
      out_shape=jax.ShapeDtypeStruct((N, D), jnp.float32),
  )(parts, wa, ba, wb, bb)


def kernel(x, edge_index, W1a, b1a, W1b, b1b, W4a, b4a, W4b, b4b):
  idx4 = jnp.stack(
      [edge_index[0].reshape(_NW, _NCHUNK, _CHUNK),
       edge_index[1].reshape(_NW, _NCHUNK, _CHUNK)], axis=2)
  zeros = jnp.zeros((N, D), jnp.float32)
  b1a2 = b1a.reshape(1, D)
  b1b2 = b1b.reshape(1, D)
  b4a2 = b4a.reshape(1, D)
  b4b2 = b4b.reshape(1, D)

  p = _sc_scatter(x, zeros, idx4).reshape(2, N, D)
  l1n = _tc_mlp_norm(p, W1a, b1a2, W1b, b1b2)
  q = _sc_scatter(l1n, zeros, idx4).reshape(2, N, D)
  return _tc_mlp(q, W4a, b4a2, W4b, b4b2)


# speedup vs baseline: 12.8398x; 1.1448x over previous
import functools

import jax
import jax.numpy as jnp
from jax import lax
from jax.experimental import pallas as pl
from jax.experimental.pallas import tpu as pltpu
from jax.experimental.pallas import tpu_sc as plsc

N = 10000
E = 320000
D = 128
NORM_SCALE = 20.0

_NC = 2
_NS = 16
_NW = _NC * _NS
_EPT = E // _NW
_CHUNK = 40
_NCHUNK = _EPT // _CHUNK
_M = 9
_DS = 3
_DG = 3
_DI = 6
assert _DI + _DS == _M and _DG < _DI
_CP_TILES = 10
_ROWS_PT = N // _CP_TILES


def _sc_scatter(feat, zeros, idx5):
  mesh = plsc.VectorSubcoreMesh(core_axis_name="c", subcore_axis_name="s")

  @functools.partial(
      pl.kernel,
      out_type=jax.ShapeDtypeStruct((2 * N, D), jnp.float32),
      mesh=mesh,
      scratch_types=[
          pltpu.VMEM((_M, _CHUNK), jnp.int32),
          pltpu.VMEM((_M, _CHUNK), jnp.int32),
          pltpu.VMEM((_M, _CHUNK, D), jnp.float32),
          pltpu.VMEM_SHARED((N, D), jnp.float32),
          pltpu.SemaphoreType.DMA,
          pltpu.SemaphoreType.DMA,
          pltpu.SemaphoreType.DMA,
      ],
  )
  def k(feat_hbm, zero_hbm, idx_hbm, out_hbm, sidxb, didxb, rows, acc,
        sem_i, sem_g, sem_s):
    c = lax.axis_index("c")
    s = lax.axis_index("s")
    wid = s * _NC + c
    r0 = s * _ROWS_PT

    def fire_idx(i, m):
      pltpu.async_copy(idx_hbm.at[0].at[wid].at[i], sidxb.at[m], sem_i)
      pltpu.async_copy(idx_hbm.at[1].at[wid].at[i], didxb.at[m], sem_i)

    def drain_idx(m):
      pltpu.make_async_copy(idx_hbm.at[0].at[0].at[0], sidxb.at[m],
                            sem_i).wait()
      pltpu.make_async_copy(idx_hbm.at[0].at[0].at[0], didxb.at[m],
                            sem_i).wait()

    def fire_gather(i, m):
      pltpu.async_copy(feat_hbm.at[sidxb.at[m]], rows.at[m], sem_g)

    def drain_gather(m):
      pltpu.make_async_copy(feat_hbm.at[pl.ds(0, _CHUNK)], rows.at[m],
                            sem_g).wait()

    def fire_scatter(i, m):
      pltpu.async_copy(rows.at[m], acc.at[didxb.at[m]], sem_s, add=True)

    def drain_scatter(m):
      pltpu.make_async_copy(feat_hbm.at[pl.ds(0, _CHUNK)], rows.at[m],
                            sem_s).wait()

    for j in range(_DI):
      fire_idx(j, j)
    for j in range(_DG):
      drain_idx(j)
      fire_gather(j, j)

    @pl.when(jnp.logical_and(s < _CP_TILES, c == 0))
    def _():
      pltpu.sync_copy(feat_hbm.at[pl.ds(r0, _ROWS_PT)],
                      acc.at[pl.ds(r0, _ROWS_PT)])

    @pl.when(jnp.logical_and(s < _CP_TILES, c != 0))
    def _():
      pltpu.sync_copy(zero_hbm.at[pl.ds(r0, _ROWS_PT)],
                      acc.at[pl.ds(r0, _ROWS_PT)])

    plsc.subcore_barrier()

    def step(i, guard_lo=True, guard_hi=True):
      m = i % _M
      if guard_lo:
        drain_scatter((i - _DS) % _M)
      if guard_hi:
        fire_idx(i + _DI, (i + _DI) % _M)
        drain_idx((i + _DG) % _M)
        fire_gather(i + _DG, (i + _DG) % _M)
      drain_gather(m)
      fire_scatter(i, m)

    for i in range(_DS):
      step(i, guard_lo=False)

    _STEADY_LO = _DS
    _STEADY_HI = _NCHUNK - _DI - ((_NCHUNK - _DI - _DS) % _M)

    def steady(t, carry):
      i0 = _STEADY_LO + t * _M
      for kk in range(_M):
        i = i0 + kk
        m = (_STEADY_LO + kk) % _M
        drain_scatter((m - _DS) % _M)
        fire_idx(i + _DI, (m + _DI) % _M)
        drain_idx((m + _DG) % _M)
        fire_gather(i + _DG, (m + _DG) % _M)
        drain_gather(m)
        fire_scatter(i, m)
      return carry

    lax.fori_loop(0, (_STEADY_HI - _STEADY_LO) // _M, steady, 0)

    for i in range(_STEADY_HI, _NCHUNK):
      m = i % _M
      drain_scatter((i - _DS) % _M)
      if i + _DI < _NCHUNK:
        fire_idx(i + _DI, (i + _DI) % _M)
      if i + _DG < _NCHUNK:
        drain_idx((i + _DG) % _M)
        fire_gather(i + _DG, (i + _DG) % _M)
      drain_gather(m)
      fire_scatter(i, m)

    for i in range(_NCHUNK - _DS, _NCHUNK):
      drain_scatter(i % _M)

    plsc.subcore_barrier()

    @pl.when(s < _CP_TILES)
    def _():
      pltpu.sync_copy(acc.at[pl.ds(r0, _ROWS_PT)],
                      out_hbm.at[pl.ds(c * N + r0, _ROWS_PT)])

  return k(feat, zeros, idx5)


_BLK = 1000


_NBLK = N // _BLK


def _mlp_norm_body(p_ref, wa_ref, ba_ref, wb_ref, bb_ref, out_ref,
                   scaled_scr, colsum_scr):
  i = pl.program_id(0)

  @pl.when(i < _NBLK)
  def _():
    h = p_ref[0] + p_ref[1]
    t = jnp.maximum(
        jnp.dot(h, wa_ref[...], preferred_element_type=jnp.float32)
        + ba_ref[...], 0.0)
    l1 = (jnp.dot(t, wb_ref[...], preferred_element_type=jnp.float32)
          + bb_ref[...])
    rn = jnp.sqrt(1e-6 + jnp.sum(l1 * l1, axis=1, keepdims=True))
    scaled_scr[pl.ds(i * _BLK, _BLK), :] = NORM_SCALE * l1 / rn
    csum = jnp.sum(l1, axis=0, keepdims=True)
    colsum_scr[...] = jnp.where(i == 0, csum, colsum_scr[...] + csum)

  @pl.when(i >= _NBLK)
  def _():
    j = i - _NBLK
    out_ref[...] = jnp.maximum(
        scaled_scr[pl.ds(j * _BLK, _BLK), :]
        - colsum_scr[...] * (1.0 / N), 0.0)


def _tc_mlp_norm(parts, wa, ba, wb, bb):
  grid = (2 * _NBLK,)
  return pl.pallas_call(
      _mlp_norm_body,
      grid=grid,
      in_specs=[
          pl.BlockSpec((2, _BLK, D), lambda i: (0, jnp.minimum(i, _NBLK - 1),
                                                0)),
          pl.BlockSpec((D, D), lambda i: (0, 0)),
          pl.BlockSpec((D,), lambda i: (0,)),
          pl.BlockSpec((D, D), lambda i: (0, 0)),
          pl.BlockSpec((D,), lambda i: (0,)),
      ],
      out_specs=pl.BlockSpec((_BLK, D),
                             lambda i: (jnp.maximum(i - _NBLK, 0), 0)),
      out_shape=jax.ShapeDtypeStruct((N, D), jnp.float32),
      scratch_shapes=[
          pltpu.VMEM((N, D), jnp.float32),
          pltpu.VMEM((1, D), jnp.float32),
      ],
  )(parts, wa, ba, wb, bb)


def _mlp_body(p_ref, wa_ref, ba_ref, wb_ref, bb_ref, out_ref):
  h = p_ref[0] + p_ref[1]
  t = jnp.maximum(
      jnp.dot(h, wa_ref[...], preferred_element_type=jnp.float32)
      + ba_ref[...], 0.0)
  out_ref[...] = (
      jnp.dot(t, wb_ref[...], preferred_element_type=jnp.float32)
      + bb_ref[...])


def _tc_mlp(parts, wa, ba, wb, bb):
  grid = (N // _BLK,)
  return pl.pallas_call(
      _mlp_body,
      grid=grid,
      in_specs=[
          pl.BlockSpec((2, _BLK, D), lambda i: (0, i, 0)),
          pl.BlockSpec((D, D), lambda i: (0, 0)),
          pl.BlockSpec((D,), lambda i: (0,)),
          pl.BlockSpec((D, D), lambda i: (0, 0)),
          pl.BlockSpec((D,), lambda i: (0,)),
      ],
      out_specs=pl.BlockSpec((_BLK, D), lambda i: (i, 0)),
      out_shape=jax.ShapeDtypeStruct((N, D), jnp.float32),
  )(parts, wa, ba, wb, bb)


def kernel(x, edge_index, W1a, b1a, W1b, b1b, W4a, b4a, W4b, b4b):
  idx5 = edge_index.reshape(2, _NW, _NCHUNK, _CHUNK)
  zeros = jnp.zeros((N, D), jnp.float32)

  p = _sc_scatter(x, zeros, idx5).reshape(2, N, D)
  l1n = _tc_mlp_norm(p, W1a, b1a, W1b, b1b)
  q = _sc_scatter(l1n, zeros, idx5).reshape(2, N, D)
  return _tc_mlp(q, W4a, b4a, W4b, b4b)


# speedup vs baseline: 12.9622x; 1.0095x over previous
import functools

import jax
import jax.numpy as jnp
from jax import lax
from jax.experimental import pallas as pl
from jax.experimental.pallas import tpu as pltpu
from jax.experimental.pallas import tpu_sc as plsc

N = 10000
E = 320000
D = 128
NORM_SCALE = 20.0

_NC = 2
_NS = 16
_NW = _NC * _NS
_TPT = 9984
_XTRA = 128
_XBASE = _NW * _TPT
_CHUNK = 32
_NCHUNK = _TPT // _CHUNK
_M = 6
_DS = 3
_DG = 3
assert _DS + _DG == _M
_CP_TILES = 10
_ROWS_PT = N // _CP_TILES


def _sc_scatter(feat, zeros, ei):
  mesh = plsc.VectorSubcoreMesh(core_axis_name="c", subcore_axis_name="s")

  @functools.partial(
      pl.kernel,
      out_type=jax.ShapeDtypeStruct((2 * N, D), jnp.float32),
      mesh=mesh,
      scratch_types=[
          pltpu.VMEM((_TPT,), jnp.int32),
          pltpu.VMEM((_TPT,), jnp.int32),
          pltpu.VMEM((_XTRA,), jnp.int32),
          pltpu.VMEM((_XTRA,), jnp.int32),
          pltpu.VMEM((_M, _CHUNK, D), jnp.float32),
          pltpu.VMEM_SHARED((N, D), jnp.float32),
          pltpu.SemaphoreType.DMA,
          pltpu.SemaphoreType.DMA,
          pltpu.SemaphoreType.DMA,
      ],
  )
  def k(feat_hbm, zero_hbm, idx_hbm, out_hbm, sidx, didx, xsidx, xdidx,
        rows, acc, sem_i, sem_g, sem_s):
    c = lax.axis_index("c")
    s = lax.axis_index("s")
    wid = s * _NC + c
    r0 = s * _ROWS_PT
    eb = wid * _TPT

    pltpu.async_copy(idx_hbm.at[0].at[pl.ds(eb, _TPT)], sidx, sem_i)
    pltpu.async_copy(idx_hbm.at[1].at[pl.ds(eb, _TPT)], didx, sem_i)

    @pl.when(wid < 4)
    def _():
      xb = _XBASE + wid * _XTRA
      pltpu.async_copy(idx_hbm.at[0].at[pl.ds(xb, _XTRA)], xsidx, sem_i)
      pltpu.async_copy(idx_hbm.at[1].at[pl.ds(xb, _XTRA)], xdidx, sem_i)

    @pl.when(jnp.logical_and(s < _CP_TILES, c == 0))
    def _():
      pltpu.sync_copy(feat_hbm.at[pl.ds(r0, _ROWS_PT)],
                      acc.at[pl.ds(r0, _ROWS_PT)])

    @pl.when(jnp.logical_and(s < _CP_TILES, c != 0))
    def _():
      pltpu.sync_copy(zero_hbm.at[pl.ds(r0, _ROWS_PT)],
                      acc.at[pl.ds(r0, _ROWS_PT)])

    pltpu.make_async_copy(idx_hbm.at[0].at[pl.ds(0, _TPT)], sidx,
                          sem_i).wait()
    pltpu.make_async_copy(idx_hbm.at[0].at[pl.ds(0, _TPT)], didx,
                          sem_i).wait()

    @pl.when(wid < 4)
    def _():
      pltpu.make_async_copy(idx_hbm.at[0].at[pl.ds(0, _XTRA)], xsidx,
                            sem_i).wait()
      pltpu.make_async_copy(idx_hbm.at[0].at[pl.ds(0, _XTRA)], xdidx,
                            sem_i).wait()

    plsc.subcore_barrier()

    def fire_gather(i, m):
      pltpu.async_copy(feat_hbm.at[sidx.at[pl.ds(i * _CHUNK, _CHUNK)]],
                       rows.at[m], sem_g)

    def drain_gather(m):
      pltpu.make_async_copy(feat_hbm.at[pl.ds(0, _CHUNK)], rows.at[m],
                            sem_g).wait()

    def fire_scatter(i, m):
      pltpu.async_copy(rows.at[m], acc.at[didx.at[pl.ds(i * _CHUNK, _CHUNK)]],
                       sem_s, add=True)

    def drain_scatter(m):
      pltpu.make_async_copy(feat_hbm.at[pl.ds(0, _CHUNK)], rows.at[m],
                            sem_s).wait()

    for j in range(_DG):
      fire_gather(j, j)

    for i in range(_DS):
      fire_gather(i + _DG, (i + _DG) % _M)
      drain_gather(i % _M)
      fire_scatter(i, i % _M)

    def steady(t, carry):
      i0 = _DS + t * _M
      for kk in range(_M):
        i = i0 + kk
        m = (_DS + kk) % _M
        drain_scatter((m + _DG) % _M)
        fire_gather(i + _DG, (m + _DG) % _M)
        drain_gather(m)
        fire_scatter(i, m)
      return carry

    _STEADY_N = (_NCHUNK - _DG - _DS) // _M
    lax.fori_loop(0, _STEADY_N, steady, 0)

    for i in range(_DS + _STEADY_N * _M, _NCHUNK):
      m = i % _M
      drain_scatter((i - _DS) % _M)
      drain_gather(m)
      fire_scatter(i, m)

    for i in range(_NCHUNK - _DS, _NCHUNK):
      drain_scatter(i % _M)

    @pl.when(wid < 4)
    def _():
      nx = _XTRA // _CHUNK
      for j in range(nx):
        pltpu.async_copy(
            feat_hbm.at[xsidx.at[pl.ds(j * _CHUNK, _CHUNK)]], rows.at[j],
            sem_g)
      for j in range(nx):
        drain_gather(j)
        pltpu.async_copy(
            rows.at[j], acc.at[xdidx.at[pl.ds(j * _CHUNK, _CHUNK)]],
            sem_s, add=True)
      for j in range(nx):
        drain_scatter(j)

    plsc.subcore_barrier()

    @pl.when(s < _CP_TILES)
    def _():
      pltpu.sync_copy(acc.at[pl.ds(r0, _ROWS_PT)],
                      out_hbm.at[pl.ds(c * N + r0, _ROWS_PT)])

  return k(feat, zeros, ei)


_BLK = 1000


_NBLK = N // _BLK


def _mlp_norm_body(p_ref, wa_ref, ba_ref, wb_ref, bb_ref, out_ref,
                   scaled_scr, colsum_scr):
  i = pl.program_id(0)

  @pl.when(i < _NBLK)
  def _():
    h = p_ref[0] + p_ref[1]
    t = jnp.maximum(
        jnp.dot(h, wa_ref[...], preferred_element_type=jnp.float32)
        + ba_ref[...], 0.0)
    l1 = (jnp.dot(t, wb_ref[...], preferred_element_type=jnp.float32)
          + bb_ref[...])
    rn = jnp.sqrt(1e-6 + jnp.sum(l1 * l1, axis=1, keepdims=True))
    scaled_scr[pl.ds(i * _BLK, _BLK), :] = NORM_SCALE * l1 / rn
    csum = jnp.sum(l1, axis=0, keepdims=True)
    colsum_scr[...] = jnp.where(i == 0, csum, colsum_scr[...] + csum)

  @pl.when(i >= _NBLK)
  def _():
    j = i - _NBLK
    out_ref[...] = jnp.maximum(
        scaled_scr[pl.ds(j * _BLK, _BLK), :]
        - colsum_scr[...] * (1.0 / N), 0.0)


def _tc_mlp_norm(parts, wa, ba, wb, bb):
  grid = (2 * _NBLK,)
  return pl.pallas_call(
      _mlp_norm_body,
      grid=grid,
      in_specs=[
          pl.BlockSpec((2, _BLK, D), lambda i: (0, jnp.minimum(i, _NBLK - 1),
                                                0)),
          pl.BlockSpec((D, D), lambda i: (0, 0)),
          pl.BlockSpec((D,), lambda i: (0,)),
          pl.BlockSpec((D, D), lambda i: (0, 0)),
          pl.BlockSpec((D,), lambda i: (0,)),
      ],
      out_specs=pl.BlockSpec((_BLK, D),
                             lambda i: (jnp.maximum(i - _NBLK, 0), 0)),
      out_shape=jax.ShapeDtypeStruct((N, D), jnp.float32),
      scratch_shapes=[
          pltpu.VMEM((N, D), jnp.float32),
          pltpu.VMEM((1, D), jnp.float32),
      ],
  )(parts, wa, ba, wb, bb)


def _mlp_body(p_ref, wa_ref, ba_ref, wb_ref, bb_ref, out_ref):
  h = p_ref[0] + p_ref[1]
  t = jnp.maximum(
      jnp.dot(h, wa_ref[...], preferred_element_type=jnp.float32)
      + ba_ref[...], 0.0)
  out_ref[...] = (
      jnp.dot(t, wb_ref[...], preferred_element_type=jnp.float32)
      + bb_ref[...])


def _tc_mlp(parts, wa, ba, wb, bb):
  grid = (N // _BLK,)
  return pl.pallas_call(
      _mlp_body,
      grid=grid,
      in_specs=[
          pl.BlockSpec((2, _BLK, D), lambda i: (0, i, 0)),
          pl.BlockSpec((D, D), lambda i: (0, 0)),
          pl.BlockSpec((D,), lambda i: (0,)),
          pl.BlockSpec((D, D), lambda i: (0, 0)),
          pl.BlockSpec((D,), lambda i: (0,)),
      ],
      out_specs=pl.BlockSpec((_BLK, D), lambda i: (i, 0)),
      out_shape=jax.ShapeDtypeStruct((N, D), jnp.float32),
  )(parts, wa, ba, wb, bb)


def kernel(x, edge_index, W1a, b1a, W1b, b1b, W4a, b4a, W4b, b4b):
  zeros = jnp.zeros((N, D), jnp.float32)

  p = _sc_scatter(x, zeros, edge_index).reshape(2, N, D)
  l1n = _tc_mlp_norm(p, W1a, b1a, W1b, b1b)
  q = _sc_scatter(l1n, zeros, edge_index).reshape(2, N, D)
  return _tc_mlp(q, W4a, b4a, W4b, b4b)


# speedup vs baseline: 13.3172x; 1.0274x over previous
import functools

import jax
import jax.numpy as jnp
from jax import lax
from jax.experimental import pallas as pl
from jax.experimental.pallas import tpu as pltpu
from jax.experimental.pallas import tpu_sc as plsc

N = 10000
E = 320000
D = 128
NORM_SCALE = 20.0

_NC = 2
_NS = 16
_NW = _NC * _NS
_TPT = 9984
_XTRA = 128
_XBASE = _NW * _TPT
_CHUNK = 48
_NCHUNK = _TPT // _CHUNK
_M = 4
_DS = 2
_DG = 2
assert _DS + _DG == _M
_XCHUNK = 32
_CP_TILES = 10
_ROWS_PT = N // _CP_TILES


def _sc_scatter(feat, zeros, ei):
  mesh = plsc.VectorSubcoreMesh(core_axis_name="c", subcore_axis_name="s")

  @functools.partial(
      pl.kernel,
      out_type=jax.ShapeDtypeStruct((2 * N, D), jnp.float32),
      mesh=mesh,
      scratch_types=[
          pltpu.VMEM((_TPT,), jnp.int32),
          pltpu.VMEM((_TPT,), jnp.int32),
          pltpu.VMEM((_XTRA,), jnp.int32),
          pltpu.VMEM((_XTRA,), jnp.int32),
          pltpu.VMEM((_M, _CHUNK, D), jnp.float32),
          pltpu.VMEM_SHARED((N, D), jnp.float32),
          pltpu.SemaphoreType.DMA,
          pltpu.SemaphoreType.DMA,
          pltpu.SemaphoreType.DMA,
      ],
  )
  def k(feat_hbm, zero_hbm, idx_hbm, out_hbm, sidx, didx, xsidx, xdidx,
        rows, acc, sem_i, sem_g, sem_s):
    c = lax.axis_index("c")
    s = lax.axis_index("s")
    wid = s * _NC + c
    r0 = s * _ROWS_PT
    eb = wid * _TPT

    pltpu.async_copy(idx_hbm.at[0].at[pl.ds(eb, _TPT)], sidx, sem_i)
    pltpu.async_copy(idx_hbm.at[1].at[pl.ds(eb, _TPT)], didx, sem_i)

    @pl.when(wid < 4)
    def _():
      xb = _XBASE + wid * _XTRA
      pltpu.async_copy(idx_hbm.at[0].at[pl.ds(xb, _XTRA)], xsidx, sem_i)
      pltpu.async_copy(idx_hbm.at[1].at[pl.ds(xb, _XTRA)], xdidx, sem_i)

    @pl.when(jnp.logical_and(s < _CP_TILES, c == 0))
    def _():
      pltpu.sync_copy(feat_hbm.at[pl.ds(r0, _ROWS_PT)],
                      acc.at[pl.ds(r0, _ROWS_PT)])

    @pl.when(jnp.logical_and(s < _CP_TILES, c != 0))
    def _():
      pltpu.sync_copy(zero_hbm.at[pl.ds(r0, _ROWS_PT)],
                      acc.at[pl.ds(r0, _ROWS_PT)])

    pltpu.make_async_copy(idx_hbm.at[0].at[pl.ds(0, _TPT)], sidx,
                          sem_i).wait()
    pltpu.make_async_copy(idx_hbm.at[0].at[pl.ds(0, _TPT)], didx,
                          sem_i).wait()

    @pl.when(wid < 4)
    def _():
      pltpu.make_async_copy(idx_hbm.at[0].at[pl.ds(0, _XTRA)], xsidx,
                            sem_i).wait()
      pltpu.make_async_copy(idx_hbm.at[0].at[pl.ds(0, _XTRA)], xdidx,
                            sem_i).wait()

    plsc.subcore_barrier()

    def fire_gather(i, m):
      pltpu.async_copy(feat_hbm.at[sidx.at[pl.ds(i * _CHUNK, _CHUNK)]],
                       rows.at[m], sem_g)

    def drain_gather(m):
      pltpu.make_async_copy(feat_hbm.at[pl.ds(0, _CHUNK)], rows.at[m],
                            sem_g).wait()

    def fire_scatter(i, m):
      pltpu.async_copy(rows.at[m], acc.at[didx.at[pl.ds(i * _CHUNK, _CHUNK)]],
                       sem_s, add=True)

    def drain_scatter(m):
      pltpu.make_async_copy(feat_hbm.at[pl.ds(0, _CHUNK)], rows.at[m],
                            sem_s).wait()

    for j in range(_DG):
      fire_gather(j, j)

    for i in range(_DS):
      fire_gather(i + _DG, (i + _DG) % _M)
      drain_gather(i % _M)
      fire_scatter(i, i % _M)

    def steady(t, carry):
      i0 = _DS + t * _M
      for kk in range(_M):
        i = i0 + kk
        m = (_DS + kk) % _M
        drain_scatter((m + _DG) % _M)
        fire_gather(i + _DG, (m + _DG) % _M)
        drain_gather(m)
        fire_scatter(i, m)
      return carry

    _STEADY_N = (_NCHUNK - _DG - _DS) // _M
    lax.fori_loop(0, _STEADY_N, steady, 0)

    for i in range(_DS + _STEADY_N * _M, _NCHUNK):
      m = i % _M
      drain_scatter((i - _DS) % _M)
      drain_gather(m)
      fire_scatter(i, m)

    for i in range(_NCHUNK - _DS, _NCHUNK):
      drain_scatter(i % _M)

    @pl.when(wid < 4)
    def _():
      nx = _XTRA // _XCHUNK
      for j in range(nx):
        pltpu.async_copy(
            feat_hbm.at[xsidx.at[pl.ds(j * _XCHUNK, _XCHUNK)]],
            rows.at[j % _M].at[pl.ds(0, _XCHUNK)], sem_g)
      for j in range(nx):
        pltpu.make_async_copy(feat_hbm.at[pl.ds(0, _XCHUNK)],
                              rows.at[j % _M].at[pl.ds(0, _XCHUNK)],
                              sem_g).wait()
        pltpu.async_copy(
            rows.at[j % _M].at[pl.ds(0, _XCHUNK)],
            acc.at[xdidx.at[pl.ds(j * _XCHUNK, _XCHUNK)]],
            sem_s, add=True)
      for j in range(nx):
        pltpu.make_async_copy(feat_hbm.at[pl.ds(0, _XCHUNK)],
                              rows.at[j % _M].at[pl.ds(0, _XCHUNK)],
                              sem_s).wait()

    plsc.subcore_barrier()

    @pl.when(s < _CP_TILES)
    def _():
      pltpu.sync_copy(acc.at[pl.ds(r0, _ROWS_PT)],
                      out_hbm.at[pl.ds(c * N + r0, _ROWS_PT)])

  return k(feat, zeros, ei)


_BLK = 1000


_NBLK = N // _BLK


def _mlp_norm_body(p_ref, wa_ref, ba_ref, wb_ref, bb_ref, out_ref,
                   scaled_scr, colsum_scr):
  i = pl.program_id(0)

  @pl.when(i < _NBLK)
  def _():
    h = p_ref[0] + p_ref[1]
    t = jnp.maximum(
        jnp.dot(h, wa_ref[...], preferred_element_type=jnp.float32)
        + ba_ref[...], 0.0)
    l1 = (jnp.dot(t, wb_ref[...], preferred_element_type=jnp.float32)
          + bb_ref[...])
    rn = jnp.sqrt(1e-6 + jnp.sum(l1 * l1, axis=1, keepdims=True))
    scaled_scr[pl.ds(i * _BLK, _BLK), :] = NORM_SCALE * l1 / rn
    csum = jnp.sum(l1, axis=0, keepdims=True)
    colsum_scr[...] = jnp.where(i == 0, csum, colsum_scr[...] + csum)

  @pl.when(i >= _NBLK)
  def _():
    j = i - _NBLK
    out_ref[...] = jnp.maximum(
        scaled_scr[pl.ds(j * _BLK, _BLK), :]
        - colsum_scr[...] * (1.0 / N), 0.0)


def _tc_mlp_norm(parts, wa, ba, wb, bb):
  grid = (2 * _NBLK,)
  return pl.pallas_call(
      _mlp_norm_body,
      grid=grid,
      in_specs=[
          pl.BlockSpec((2, _BLK, D), lambda i: (0, jnp.minimum(i, _NBLK - 1),
                                                0)),
          pl.BlockSpec((D, D), lambda i: (0, 0)),
          pl.BlockSpec((D,), lambda i: (0,)),
          pl.BlockSpec((D, D), lambda i: (0, 0)),
          pl.BlockSpec((D,), lambda i: (0,)),
      ],
      out_specs=pl.BlockSpec((_BLK, D),
                             lambda i: (jnp.maximum(i - _NBLK, 0), 0)),
      out_shape=jax.ShapeDtypeStruct((N, D), jnp.float32),
      scratch_shapes=[
          pltpu.VMEM((N, D), jnp.float32),
          pltpu.VMEM((1, D), jnp.float32),
      ],
  )(parts, wa, ba, wb, bb)


def _mlp_body(p_ref, wa_ref, ba_ref, wb_ref, bb_ref, out_ref):
  h = p_ref[0] + p_ref[1]
  t = jnp.maximum(
      jnp.dot(h, wa_ref[...], preferred_element_type=jnp.float32)
      + ba_ref[...], 0.0)
  out_ref[...] = (
      jnp.dot(t, wb_ref[...], preferred_element_type=jnp.float32)
      + bb_ref[...])


def _tc_mlp(parts, wa, ba, wb, bb):
  grid = (N // _BLK,)
  return pl.pallas_call(
      _mlp_body,
      grid=grid,
      in_specs=[
          pl.BlockSpec((2, _BLK, D), lambda i: (0, i, 0)),
          pl.BlockSpec((D, D), lambda i: (0, 0)),
          pl.BlockSpec((D,), lambda i: (0,)),
          pl.BlockSpec((D, D), lambda i: (0, 0)),
          pl.BlockSpec((D,), lambda i: (0,)),
      ],
      out_specs=pl.BlockSpec((_BLK, D), lambda i: (i, 0)),
      out_shape=jax.ShapeDtypeStruct((N, D), jnp.float32),
  )(parts, wa, ba, wb, bb)


def kernel(x, edge_index, W1a, b1a, W1b, b1b, W4a, b4a, W4b, b4b):
  zeros = jnp.zeros((N, D), jnp.float32)

  p = _sc_scatter(x, zeros, edge_index).reshape(2, N, D)
  l1n = _tc_mlp_norm(p, W1a, b1a, W1b, b1b)
  q = _sc_scatter(l1n, zeros, edge_index).reshape(2, N, D)
  return _tc_mlp(q, W4a, b4a, W4b, b4b)
